# baseline (speedup 1.0000x reference)
import functools

import jax
import jax.numpy as jnp
import numpy as np
from jax import lax
from jax.experimental import pallas as pl
from jax.experimental.pallas import tpu as pltpu
from jax.experimental.pallas import tpu_sc as plsc

DROP_P = 0.1
KEY_LO = 42
KEY_HI = 0
T = 2048

N_WORKERS = 32
ROWS = 8192
RPW = ROWS // N_WORKERS
KCHUNK = 8
NCHUNK = 5


def _np_rotl(x, r):
    return ((x << np.uint32(r)) | (x >> np.uint32(32 - r))).astype(np.uint32)


def _np_dropped_mask():
    i = np.arange(ROWS, dtype=np.uint32)
    k0, k1 = np.uint32(KEY_HI), np.uint32(KEY_LO)
    ks = [k0, k1, np.uint32(k0 ^ k1 ^ np.uint32(0x1BD11BDA))]
    x0 = np.zeros_like(i) + ks[0]
    x1 = (i + ks[1]).astype(np.uint32)
    rotations = [(13, 15, 26, 6), (17, 29, 16, 24)]
    for rnd in range(5):
        for r in rotations[rnd % 2]:
            x0 = (x0 + x1).astype(np.uint32)
            x1 = _np_rotl(x1, r)
            x1 = (x1 ^ x0).astype(np.uint32)
        x0 = (x0 + ks[(rnd + 1) % 3]).astype(np.uint32)
        x1 = (x1 + ks[(rnd + 2) % 3] + np.uint32(rnd + 1)).astype(np.uint32)
    bits = x0 ^ x1
    u = ((bits >> np.uint32(9)) | np.uint32(0x3F800000)).view(np.float32)
    u = u - np.float32(1.0)
    drop = u < np.float32(DROP_P)
    drop[i % np.uint32(T) == 0] = False
    return drop


def _dropped_table():
    drop = _np_dropped_mask()
    tbl = np.zeros((N_WORKERS, NCHUNK * KCHUNK), dtype=np.int32)
    for w in range(N_WORKERS):
        ids = np.nonzero(drop[w * RPW:(w + 1) * RPW])[0] + w * RPW
        assert 1 <= len(ids) <= NCHUNK * KCHUNK
        pad = np.full(NCHUNK * KCHUNK - len(ids), ids[0], dtype=np.int32)
        tbl[w] = np.concatenate([ids.astype(np.int32), pad])
    return tbl.reshape(N_WORKERS, NCHUNK, KCHUNK)


def _sc_body(x_hbm, tbl_hbm, out_hbm, idx_v, zero_v):
    c = lax.axis_index("c")
    s = lax.axis_index("s")
    wid = s * 2 + c
    base = wid * RPW

    z = jnp.zeros((16,), jnp.float32)

    def zbody(j, carry):
        for r in range(KCHUNK):
            zero_v[r, pl.ds(j * 16, 16)] = z
        return carry

    lax.fori_loop(0, zero_v.shape[1] // 16, zbody, 0)

    pltpu.sync_copy(tbl_hbm.at[wid], idx_v)
    pltpu.sync_copy(x_hbm.at[pl.ds(base, RPW)], out_hbm.at[pl.ds(base, RPW)])
    for j in range(NCHUNK):
        pltpu.sync_copy(zero_v, out_hbm.at[idx_v.at[j]])


def kernel(x):
    B, t, F = x.shape
    x2 = x.reshape(ROWS, F)
    tbl = jnp.asarray(_dropped_table())
    mesh = plsc.VectorSubcoreMesh(core_axis_name="c", subcore_axis_name="s")
    sc_k = functools.partial(
        pl.kernel,
        mesh=mesh,
        out_type=jax.ShapeDtypeStruct((ROWS, F), jnp.float32),
        scratch_types=[
            pltpu.VMEM((NCHUNK, KCHUNK), jnp.int32),
            pltpu.VMEM((KCHUNK, F), jnp.float32),
        ],
    )(_sc_body)
    out = sc_k(x2, tbl)
    return out.reshape(B, t, F)


# speedup vs baseline: 32.3789x; 32.3789x over previous
import functools

import jax
import jax.numpy as jnp
import numpy as np
from jax import lax
from jax.experimental import pallas as pl
from jax.experimental.pallas import tpu as pltpu
from jax.experimental.pallas import tpu_sc as plsc

DROP_P = 0.1
KEY_LO = 42
KEY_HI = 0
T = 2048

N_WORKERS = 32
ROWS = 8192
RPW = ROWS // N_WORKERS
KCHUNK = 8
NCHUNK = 5


def _np_rotl(x, r):
    return ((x << np.uint32(r)) | (x >> np.uint32(32 - r))).astype(np.uint32)


def _np_dropped_mask():
    i = np.arange(ROWS, dtype=np.uint32)
    k0, k1 = np.uint32(KEY_HI), np.uint32(KEY_LO)
    ks = [k0, k1, np.uint32(k0 ^ k1 ^ np.uint32(0x1BD11BDA))]
    x0 = np.zeros_like(i) + ks[0]
    x1 = (i + ks[1]).astype(np.uint32)
    rotations = [(13, 15, 26, 6), (17, 29, 16, 24)]
    for rnd in range(5):
        for r in rotations[rnd % 2]:
            x0 = (x0 + x1).astype(np.uint32)
            x1 = _np_rotl(x1, r)
            x1 = (x1 ^ x0).astype(np.uint32)
        x0 = (x0 + ks[(rnd + 1) % 3]).astype(np.uint32)
        x1 = (x1 + ks[(rnd + 2) % 3] + np.uint32(rnd + 1)).astype(np.uint32)
    bits = x0 ^ x1
    u = ((bits >> np.uint32(9)) | np.uint32(0x3F800000)).view(np.float32)
    u = u - np.float32(1.0)
    drop = u < np.float32(DROP_P)
    drop[i % np.uint32(T) == 0] = False
    return drop


def _dropped_table():
    drop = _np_dropped_mask()
    tbl = np.zeros((N_WORKERS, NCHUNK * KCHUNK), dtype=np.int32)
    for w in range(N_WORKERS):
        ids = np.nonzero(drop[w * RPW:(w + 1) * RPW])[0] + w * RPW
        assert 1 <= len(ids) <= NCHUNK * KCHUNK
        pad = np.full(NCHUNK * KCHUNK - len(ids), ids[0], dtype=np.int32)
        tbl[w] = np.concatenate([ids.astype(np.int32), pad])
    return tbl.reshape(N_WORKERS, NCHUNK, KCHUNK)


CH = 8
NBUF = 2


def _sc_body(x_hbm, tbl_hbm, out_hbm, idx_v, zero_v, buf_v,
             sem_in0, sem_in1, sem_out0, sem_out1):
    c = lax.axis_index("c")
    s = lax.axis_index("s")
    wid = s * 2 + c
    base = wid * RPW
    sems_in = (sem_in0, sem_in1)
    sems_out = (sem_out0, sem_out1)

    def rows_at(i):
        return pl.ds(base + i * CH, CH)

    def obody(o, carry):
        for b in range(NBUF):
            i = o * NBUF + b

            @pl.when(o > 0)
            def _():
                pltpu.make_async_copy(
                    buf_v.at[b], out_hbm.at[rows_at(i)], sems_out[b]
                ).wait()

            pltpu.make_async_copy(
                x_hbm.at[rows_at(i)], buf_v.at[b], sems_in[b]
            ).start()
        for b in range(NBUF):
            i = o * NBUF + b
            pltpu.make_async_copy(
                x_hbm.at[rows_at(i)], buf_v.at[b], sems_in[b]
            ).wait()
            pltpu.make_async_copy(
                buf_v.at[b], out_hbm.at[rows_at(i)], sems_out[b]
            ).start()
        return carry

    z = jnp.zeros((16,), jnp.float32)

    def zbody(j, carry):
        for r in range(KCHUNK):
            zero_v[r, pl.ds(j * 16, 16)] = z
        return carry

    lax.fori_loop(0, zero_v.shape[1] // 16, zbody, 0)
    pltpu.sync_copy(tbl_hbm.at[wid], idx_v)

    lax.fori_loop(0, RPW // (CH * NBUF), obody, 0)
    for b in range(NBUF):
        pltpu.make_async_copy(
            buf_v.at[b], out_hbm.at[rows_at(b)], sems_out[b]
        ).wait()

    for j in range(NCHUNK):
        pltpu.sync_copy(zero_v, out_hbm.at[idx_v.at[j]])


def kernel(x):
    B, t, F = x.shape
    x2 = x.reshape(ROWS, F)
    tbl = jnp.asarray(_dropped_table())
    mesh = plsc.VectorSubcoreMesh(core_axis_name="c", subcore_axis_name="s")
    sc_k = functools.partial(
        pl.kernel,
        mesh=mesh,
        out_type=jax.ShapeDtypeStruct((ROWS, F), jnp.float32),
        scratch_types=[
            pltpu.VMEM((NCHUNK, KCHUNK), jnp.int32),
            pltpu.VMEM((KCHUNK, F), jnp.float32),
            pltpu.VMEM((NBUF, CH, F), jnp.float32),
            pltpu.SemaphoreType.DMA,
            pltpu.SemaphoreType.DMA,
            pltpu.SemaphoreType.DMA,
            pltpu.SemaphoreType.DMA,
        ],
    )(_sc_body)
    out = sc_k(x2, tbl)
    return out.reshape(B, t, F)


# speedup vs baseline: 47.8006x; 1.4763x over previous
import functools

import jax
import jax.numpy as jnp
from jax.experimental import pallas as pl
from jax.experimental.pallas import tpu as pltpu

DROP_P = 0.1
KEY_LO = 42
KEY_HI = 0
T = 2048

_ROT_A = (13, 15, 26, 6)
_ROT_B = (17, 29, 16, 24)


def _rotl(x, r):
    return (x << jnp.uint32(r)) | (x >> jnp.uint32(32 - r))


def _threefry2x32(x0, x1):
    k0 = jnp.uint32(KEY_HI)
    k1 = jnp.uint32(KEY_LO)
    ks = (k0, k1, k0 ^ k1 ^ jnp.uint32(0x1BD11BDA))
    x0 = x0 + ks[0]
    x1 = x1 + ks[1]
    for i in range(5):
        for r in (_ROT_A, _ROT_B)[i % 2]:
            x0 = x0 + x1
            x1 = _rotl(x1, r)
            x1 = x1 ^ x0
        x0 = x0 + ks[(i + 1) % 3]
        x1 = x1 + ks[(i + 2) % 3] + jnp.uint32(i + 1)
    return x0, x1


def _keep_multiplier(rows, lanes):
    sub = jax.lax.broadcasted_iota(jnp.uint32, (rows, lanes), 0)
    lane = jax.lax.broadcasted_iota(jnp.uint32, (rows, lanes), 1)
    i_global = sub * jnp.uint32(lanes) + lane
    a, b = _threefry2x32(jnp.zeros_like(i_global), i_global)
    bits = a ^ b
    u = jax.lax.bitcast_convert_type(
        (bits >> jnp.uint32(9)) | jnp.uint32(0x3F800000), jnp.float32
    ) - jnp.float32(1.0)
    dropped = u < jnp.float32(DROP_P)
    first_t = (i_global % jnp.uint32(T)) == jnp.uint32(0)
    keep = (~dropped) | first_t
    return jnp.where(keep, jnp.float32(1.0), jnp.float32(0.0))


def _body(rpb, x_ref, o_ref, m_scr):
    @pl.when(pl.program_id(0) == 0)
    def _():
        m_scr[:, :, 0] = _keep_multiplier(*m_scr.shape[:2])

    base = pl.program_id(0) * rpb
    o_ref[...] = x_ref[...] * m_scr[pl.ds(base, rpb), :, :]


def kernel(x):
    B, t, F = x.shape
    rows = B * t
    xr = x.reshape(rows // 128, 128, F)
    rpb = 2
    grid = (xr.shape[0] // rpb,)
    out = pl.pallas_call(
        functools.partial(_body, rpb),
        grid=grid,
        in_specs=[pl.BlockSpec((rpb, 128, F), lambda i: (i, 0, 0))],
        out_specs=pl.BlockSpec((rpb, 128, F), lambda i: (i, 0, 0)),
        out_shape=jax.ShapeDtypeStruct(xr.shape, x.dtype),
        scratch_shapes=[pltpu.VMEM((rows // 128, 128, 1), jnp.float32)],
    )(xr)
    return out.reshape(B, t, F)


# speedup vs baseline: 50.5009x; 1.0565x over previous
import jax
import jax.numpy as jnp
import numpy as np
from jax.experimental import pallas as pl
from jax.experimental.pallas import tpu as pltpu

DROP_P = 0.1
KEY_LO = 42
KEY_HI = 0
T = 2048

ROWS = 8192
BR = 1024
NBUF = 3


def _np_rotl(x, r):
    return ((x << np.uint32(r)) | (x >> np.uint32(32 - r))).astype(np.uint32)


def _np_dropped_mask():
    i = np.arange(ROWS, dtype=np.uint32)
    k0, k1 = np.uint32(KEY_HI), np.uint32(KEY_LO)
    ks = [k0, k1, np.uint32(k0 ^ k1 ^ np.uint32(0x1BD11BDA))]
    x0 = np.zeros_like(i) + ks[0]
    x1 = (i + ks[1]).astype(np.uint32)
    rotations = [(13, 15, 26, 6), (17, 29, 16, 24)]
    for rnd in range(5):
        for r in rotations[rnd % 2]:
            x0 = (x0 + x1).astype(np.uint32)
            x1 = _np_rotl(x1, r)
            x1 = (x1 ^ x0).astype(np.uint32)
        x0 = (x0 + ks[(rnd + 1) % 3]).astype(np.uint32)
        x1 = (x1 + ks[(rnd + 2) % 3] + np.uint32(rnd + 1)).astype(np.uint32)
    bits = x0 ^ x1
    u = ((bits >> np.uint32(9)) | np.uint32(0x3F800000)).view(np.float32)
    u = u - np.float32(1.0)
    drop = u < np.float32(DROP_P)
    drop[i % np.uint32(T) == 0] = False
    return drop


_DROPPED_BY_BLOCK = [
    np.nonzero(_np_dropped_mask()[k * BR:(k + 1) * BR])[0].tolist()
    for k in range(ROWS // BR)
]


def _body(x_hbm, o_hbm, buf, sem_in0, sem_in1, sem_in2,
          sem_out0, sem_out1, sem_out2):
    F = x_hbm.shape[1]
    sems_in = (sem_in0, sem_in1, sem_in2)
    sems_out = (sem_out0, sem_out1, sem_out2)
    nblocks = ROWS // BR
    zrow = jnp.zeros((1, F), jnp.float32)

    def in_copy(k):
        b = k % NBUF
        return pltpu.make_async_copy(
            x_hbm.at[pl.ds(k * BR, BR)], buf.at[b], sems_in[b]
        )

    def out_copy(k):
        b = k % NBUF
        return pltpu.make_async_copy(
            buf.at[b], o_hbm.at[pl.ds(k * BR, BR)], sems_out[b]
        )

    for j in range(NBUF):
        in_copy(j).start()
    for k in range(nblocks):
        b = k % NBUF
        in_copy(k).wait()
        for r in _DROPPED_BY_BLOCK[k]:
            buf[b, r:r + 1, :] = zrow
        out_copy(k).start()
        if k + NBUF < nblocks:
            out_copy(k).wait()
            in_copy(k + NBUF).start()
    for k in range(nblocks - NBUF, nblocks):
        out_copy(k).wait()


def kernel(x):
    B, t, F = x.shape
    x2 = x.reshape(ROWS, F)
    out = pl.pallas_call(
        _body,
        in_specs=[pl.BlockSpec(memory_space=pltpu.MemorySpace.HBM)],
        out_specs=pl.BlockSpec(memory_space=pltpu.MemorySpace.HBM),
        out_shape=jax.ShapeDtypeStruct((ROWS, F), jnp.float32),
        scratch_shapes=[
            pltpu.VMEM((NBUF, BR, F), jnp.float32),
            pltpu.SemaphoreType.DMA,
            pltpu.SemaphoreType.DMA,
            pltpu.SemaphoreType.DMA,
            pltpu.SemaphoreType.DMA,
            pltpu.SemaphoreType.DMA,
            pltpu.SemaphoreType.DMA,
        ],
    )(x2)
    return out.reshape(B, t, F)
